# baseline (speedup 1.0000x reference)
import functools

import jax
import jax.numpy as jnp
from jax import lax
from jax.experimental import pallas as pl
from jax.experimental.pallas import tpu as pltpu
from jax.experimental.pallas import tpu_sc as plsc

N = 10000
E = 320000
D_IN = 128
D_HID = 256
D_OUT = 128

NC = 2
NS = 16
NW = NC * NS
CHUNK = 128
NP = 10112
ROWS_PER_TILE = NP // NS
CPT = 80
IB = 8
E_PAD = NW * CPT * CHUNK


def _make_segsum(D):
    mesh = plsc.VectorSubcoreMesh(core_axis_name="c", subcore_axis_name="s")

    @functools.partial(
        pl.kernel,
        mesh=mesh,
        compiler_params=pltpu.CompilerParams(use_tc_tiling_on_sc=False),
        out_type=jax.ShapeDtypeStruct((NC * NP, D), jnp.float32),
        scratch_types=[
            pltpu.VMEM_SHARED((NP, D), jnp.float32),
            pltpu.VMEM((IB, CHUNK), jnp.int32),
            pltpu.VMEM((IB, CHUNK), jnp.int32),
            pltpu.VMEM((CHUNK, D), jnp.float32),
            pltpu.VMEM((CHUNK, D), jnp.float32),
            pltpu.SemaphoreType.DMA,
            pltpu.SemaphoreType.DMA,
        ],
    )
    def segsum(table, src2, dst2, out, acc, sidx, didx, rows0, rows1,
               semg0, semg1):
        c = lax.axis_index("c")
        s = lax.axis_index("s")
        wid = c * NS + s

        def zero_row(i, carry):
            for j in range(D // 16):
                rows0[i, pl.ds(j * 16, 16)] = jnp.zeros((16,), jnp.float32)
            return carry

        lax.fori_loop(0, CHUNK, zero_row, 0)
        row0 = s * ROWS_PER_TILE
        for k in range(4):
            pltpu.sync_copy(rows0, acc.at[pl.ds(row0 + k * CHUNK, CHUNK)])
        pltpu.sync_copy(rows0.at[pl.ds(0, ROWS_PER_TILE - 4 * CHUNK)],
                        acc.at[pl.ds(row0 + 4 * CHUNK,
                                     ROWS_PER_TILE - 4 * CHUNK)])
        plsc.subcore_barrier()

        def gather(j, buf, sem):
            pltpu.async_copy(table.at[sidx.at[j]], buf, sem)

        def gwait(j, buf, sem):
            pltpu.make_async_copy(table.at[sidx.at[j]], buf, sem).wait()

        def scat(j, buf):
            pltpu.sync_copy(buf, acc.at[didx.at[j]], add=True)

        def block(b, carry):
            c0 = wid * CPT + b * IB
            pltpu.sync_copy(src2.at[pl.ds(c0, IB)], sidx)
            pltpu.sync_copy(dst2.at[pl.ds(c0, IB)], didx)
            gather(0, rows0, semg0)
            for jb in range(0, IB, 2):
                gather(jb + 1, rows1, semg1)
                gwait(jb, rows0, semg0)
                scat(jb, rows0)
                if jb + 2 < IB:
                    gather(jb + 2, rows0, semg0)
                gwait(jb + 1, rows1, semg1)
                scat(jb + 1, rows1)
            return carry

        lax.fori_loop(0, CPT // IB, block, 0)
        plsc.subcore_barrier()

        for k in range(4):
            r = row0 + k * CHUNK
            pltpu.sync_copy(acc.at[pl.ds(r, CHUNK)], rows0)
            pltpu.sync_copy(rows0, out.at[pl.ds(c * NP + r, CHUNK)])
        rem = ROWS_PER_TILE - 4 * CHUNK
        r = row0 + 4 * CHUNK
        pltpu.sync_copy(acc.at[pl.ds(r, rem)], rows0.at[pl.ds(0, rem)])
        pltpu.sync_copy(rows0.at[pl.ds(0, rem)], out.at[pl.ds(c * NP + r, rem)])

    return segsum


_segsum144 = _make_segsum(D_IN + 16)
_segsum128 = _make_segsum(D_OUT)

_R = 1000


def _dense0_body(x_ref, a0_ref, a1_ref, ws0_ref, wn0_ref, b0_ref, wn1_ref,
                 h_ref, p_ref):
    agg = a0_ref[:, :D_IN] + a1_ref[:, :D_IN]
    deg = a0_ref[:, D_IN:D_IN + 16] + a1_ref[:, D_IN:D_IN + 16]
    invd = 1.0 / jnp.clip(deg[:, :1], 1.0, None)
    nb = agg * invd
    h = x_ref[...] @ ws0_ref[...] + nb @ wn0_ref[...] + b0_ref[...]
    h = jnp.maximum(h, 0.0)
    h_ref[...] = h
    p_ref[...] = h @ wn1_ref[...]


def _dense0(x, a0, a1, Ws0, Wn0, b0, Wn1):
    D0 = D_IN + 16
    return pl.pallas_call(
        _dense0_body,
        grid=(N // _R,),
        in_specs=[
            pl.BlockSpec((_R, D_IN), lambda i: (i, 0)),
            pl.BlockSpec((_R, D0), lambda i: (i, 0)),
            pl.BlockSpec((_R, D0), lambda i: (i, 0)),
            pl.BlockSpec((D_IN, D_HID), lambda i: (0, 0)),
            pl.BlockSpec((D_IN, D_HID), lambda i: (0, 0)),
            pl.BlockSpec((1, D_HID), lambda i: (0, 0)),
            pl.BlockSpec((D_HID, D_OUT), lambda i: (0, 0)),
        ],
        out_specs=[
            pl.BlockSpec((_R, D_HID), lambda i: (i, 0)),
            pl.BlockSpec((_R, D_OUT), lambda i: (i, 0)),
        ],
        out_shape=[
            jax.ShapeDtypeStruct((N, D_HID), jnp.float32),
            jax.ShapeDtypeStruct((N, D_OUT), jnp.float32),
        ],
    )(x, a0, a1, Ws0, Wn0, b0, Wn1)


def _dense1_body(h_ref, a0_ref, a1_ref, d0_ref, d1_ref, ws1_ref, b1_ref,
                 o_ref):
    deg = d0_ref[:, :1] + d1_ref[:, :1]
    invd = 1.0 / jnp.clip(deg, 1.0, None)
    nb = (a0_ref[...] + a1_ref[...]) * invd
    o_ref[...] = h_ref[...] @ ws1_ref[...] + nb + b1_ref[...]


def _dense1(h, a0, a1, d0, d1, Ws1, b1):
    return pl.pallas_call(
        _dense1_body,
        grid=(N // _R,),
        in_specs=[
            pl.BlockSpec((_R, D_HID), lambda i: (i, 0)),
            pl.BlockSpec((_R, D_OUT), lambda i: (i, 0)),
            pl.BlockSpec((_R, D_OUT), lambda i: (i, 0)),
            pl.BlockSpec((_R, 16), lambda i: (i, 0)),
            pl.BlockSpec((_R, 16), lambda i: (i, 0)),
            pl.BlockSpec((D_HID, D_OUT), lambda i: (0, 0)),
            pl.BlockSpec((1, D_OUT), lambda i: (0, 0)),
        ],
        out_specs=pl.BlockSpec((_R, D_OUT), lambda i: (i, 0)),
        out_shape=jax.ShapeDtypeStruct((N, D_OUT), jnp.float32),
    )(h, a0, a1, d0, d1, Ws1, b1)


def kernel(x, edge_index, W_self0, W_neigh0, b0, W_self1, W_neigh1, b1):
    src = edge_index[0]
    dst = edge_index[1]
    pad = E_PAD - E
    src2 = jnp.concatenate(
        [src, jnp.zeros((pad,), jnp.int32)]).reshape(-1, CHUNK)
    dst2 = jnp.concatenate(
        [dst, jnp.full((pad,), NP - 1, jnp.int32)]).reshape(-1, CHUNK)
    x_ext = jnp.concatenate(
        [x, jnp.ones((N, 16), jnp.float32)], axis=1)
    parts0 = _segsum144(x_ext, src2, dst2)
    a0, a1 = parts0[:N], parts0[NP:NP + N]
    h, p = _dense0(x, a0, a1, W_self0, W_neigh0,
                   b0.reshape(1, -1), W_neigh1)
    parts1 = _segsum128(p, src2, dst2)
    out = _dense1(h, parts1[:N], parts1[NP:NP + N],
                  a0[:, D_IN:D_IN + 16], a1[:, D_IN:D_IN + 16],
                  W_self1, b1.reshape(1, -1))
    return out
